# baseline (speedup 1.0000x reference)
import functools

import jax
import jax.numpy as jnp
from jax import lax
from jax.experimental import pallas as pl
from jax.experimental.pallas import tpu as pltpu
from jax.experimental.pallas import tpu_sc as plsc

N = 10000
NP = 10240
D = 128
NC = 2
NS = 16
NW = NC * NS
CHUNK = 64
CHT = 160
QC = 40
Q0 = 2
Q1 = 6
TOTALC = NW * CHT
EP = NW * CHT * CHUNK
TS = NP // NS

_mesh = plsc.VectorSubcoreMesh(core_axis_name="c", subcore_axis_name="s")


@functools.partial(
    pl.kernel,
    out_type=jax.ShapeDtypeStruct((NC, NP), jnp.float32),
    mesh=_mesh,
    scratch_types=[
        pltpu.VMEM_SHARED((NP,), jnp.float32),
        pltpu.VMEM((CHT, 2, CHUNK), jnp.int32),
        pltpu.VMEM((CHT, CHUNK), jnp.float32),
        pltpu.VMEM((TS,), jnp.float32),
    ] + [pltpu.SemaphoreType.DMA for _ in range(8)],
)
def _deg_kernel(e_h, w_h, out_h, acc, ebuf, wbuf, zb,
                d0, d1, d2, d3, d4, d5, d6, d7):
    c = lax.axis_index("c")
    s = lax.axis_index("s")
    wid = c * NS + s
    dsem = [d0, d1, d2, d3, d4, d5, d6, d7]

    def _z(i, carry):
        zb[pl.ds(i * 16, 16)] = jnp.zeros((16,), jnp.float32)
        return carry

    lax.fori_loop(0, TS // 16, _z, 0)
    pltpu.sync_copy(zb, acc.at[pl.ds(s * TS, TS)])
    pltpu.sync_copy(e_h.at[pl.ds(wid * CHT, CHT)], ebuf)
    pltpu.sync_copy(w_h.at[pl.ds(wid * CHT, CHT)], wbuf)
    plsc.subcore_barrier()

    def _fire(l, b):
        pltpu.async_copy(wbuf.at[l], acc.at[ebuf.at[l, 1]], dsem[b], add=True)

    def _drain(l, b):
        pltpu.make_async_copy(wbuf.at[l], acc.at[ebuf.at[l, 1]], dsem[b]).wait()

    for b in range(8):
        _fire(b, b)

    def _grp(g, carry):
        for b in range(8):
            l = 8 + g * 8 + b
            _drain(l - 8, b)
            _fire(l, b)
        return carry

    lax.fori_loop(0, (CHT - 8) // 8, _grp, 0)
    for b in range(8):
        _drain(CHT - 8 + b, b)
    plsc.subcore_barrier()
    sl = pl.ds(s * TS, TS)
    pltpu.sync_copy(acc.at[sl], out_h.at[c, sl])


NBUF = 4


@functools.partial(
    pl.kernel,
    out_type=jax.ShapeDtypeStruct((NC, NP, D), jnp.float32),
    mesh=_mesh,
    scratch_types=[
        pltpu.VMEM_SHARED((NP, D), jnp.float32),
        pltpu.VMEM((QC, 2, CHUNK), jnp.int32),
        pltpu.VMEM((QC, CHUNK), jnp.float32),
    ] + [pltpu.VMEM((CHUNK, D), jnp.float32) for _ in range(NBUF)]
      + [pltpu.SemaphoreType.DMA for _ in range(2 * NBUF)],
)
def _msg_kernel(g_h, e_h, w_h, out_h, acc, idxq, wq, r0, r1, r2, r3,
                g0, g1, g2, g3, s0, s1, s2, s3):
    c = lax.axis_index("c")
    s = lax.axis_index("s")
    nq = jnp.where(c == 0, Q0, Q1)
    qb = jnp.where(c == 0, s * Q0, NS * Q0 + s * Q1)
    rows = [r0, r1, r2, r3]
    gsem = [g0, g1, g2, g3]
    ssem = [s0, s1, s2, s3]

    def _z(i, carry):
        r0[i // 8, pl.ds((i % 8) * 16, 16)] = jnp.zeros((16,), jnp.float32)
        return carry

    lax.fori_loop(0, CHUNK * 8, _z, 0)
    for r in range(TS // CHUNK):
        pltpu.sync_copy(r0, acc.at[pl.ds(s * TS + r * CHUNK, CHUNK)])
    plsc.subcore_barrier()

    def _g_start(l, b):
        pltpu.async_copy(g_h.at[idxq.at[l, 0]], rows[b], gsem[b])

    def _g_wait(l, b):
        pltpu.make_async_copy(g_h.at[idxq.at[l, 0]], rows[b], gsem[b]).wait()

    def _s_start(l, b):
        pltpu.async_copy(rows[b], acc.at[idxq.at[l, 1]], ssem[b], add=True)

    def _s_wait(l, b):
        pltpu.make_async_copy(rows[b], acc.at[idxq.at[l, 1]], ssem[b]).wait()

    def _scale(l, b):
        rb = rows[b]

        def body(g16, c2):
            w16 = wq[l, pl.ds(g16 * 16, 16)]
            for i in range(16):
                co = w16[i]
                e = g16 * 16 + i
                for k in range(8):
                    rb[e, pl.ds(k * 16, 16)] = rb[e, pl.ds(k * 16, 16)] * co
            return c2

        lax.fori_loop(0, CHUNK // 16, body, 0)

    def _quarter(q, carry):
        base = (qb + q) * QC
        pltpu.sync_copy(e_h.at[pl.ds(base, QC)], idxq)
        pltpu.sync_copy(w_h.at[pl.ds(base, QC)], wq)
        _g_start(0, 0)
        _g_start(1, 1)
        for l in (0, 1):
            _g_start(l + 2, l + 2)
            _g_wait(l, l)
            _scale(l, l)
            _s_start(l, l)

        def _group(g, c2):
            for b in range(NBUF):
                l = 2 + g * NBUF + b
                cur = (2 + b) % NBUF
                tgt = b
                _s_wait(l - 2, tgt)
                _g_start(l + 2, tgt)
                _g_wait(l, cur)
                _scale(l, cur)
                _s_start(l, cur)
            return c2

        lax.fori_loop(0, (QC - 4) // NBUF, _group, 0)
        for l in (QC - 2, QC - 1):
            cur = l % NBUF
            _g_wait(l, cur)
            _scale(l, cur)
            _s_start(l, cur)
        for l in (QC - 4, QC - 3, QC - 2, QC - 1):
            _s_wait(l, l % NBUF)
        return carry

    lax.fori_loop(0, nq, _quarter, 0)

    plsc.subcore_barrier()
    for r in range(TS // CHUNK):
        sl = pl.ds(s * TS + r * CHUNK, CHUNK)
        pltpu.sync_copy(acc.at[sl], out_h.at[c, sl])


BLK = 1024
GRID = NP // BLK


def _tc1_body(d0, d1, x, w1, h1, g1, dis, sn):
    deg = d0[...] + d1[...] + 1.0
    di = lax.rsqrt(deg)
    dis[...] = di
    sn[...] = 1.0 / deg
    h = jnp.dot(x[...], w1[...], preferred_element_type=jnp.float32)
    h1[...] = h
    g1[...] = h * di


def _tc2_body(p0, p1, dis, sn, h1, b1, w2, h2, g2):
    z = jnp.maximum(
        (p0[...] + p1[...]) * dis[...] + h1[...] * sn[...] + b1[...], 0.0)
    h = jnp.dot(z, w2[...], preferred_element_type=jnp.float32)
    h2[...] = h
    g2[...] = h * dis[...]


def _tc3_body(q0, q1, dis, sn, h2, b2, gm, bt, out):
    y = (q0[...] + q1[...]) * dis[...] + h2[...] * sn[...] + b2[...]
    mu = jnp.mean(y, axis=-1, keepdims=True)
    yc = y - mu
    var = jnp.mean(yc * yc, axis=-1, keepdims=True)
    out[...] = yc * lax.rsqrt(var + 1e-5) * gm[...] + bt[...]


def _row_spec():
    return pl.BlockSpec((BLK, D), lambda i: (i, 0))


def _col_spec():
    return pl.BlockSpec((BLK, 1), lambda i: (i, 0))


def _full_spec():
    return pl.BlockSpec((D, D), lambda i: (0, 0))


def _vec_spec():
    return pl.BlockSpec((1, D), lambda i: (0, 0))


_tc1 = pl.pallas_call(
    _tc1_body,
    grid=(GRID,),
    in_specs=[_col_spec(), _col_spec(), _row_spec(), _full_spec()],
    out_specs=[_row_spec(), _row_spec(), _col_spec(), _col_spec()],
    out_shape=[
        jax.ShapeDtypeStruct((NP, D), jnp.float32),
        jax.ShapeDtypeStruct((NP, D), jnp.float32),
        jax.ShapeDtypeStruct((NP, 1), jnp.float32),
        jax.ShapeDtypeStruct((NP, 1), jnp.float32),
    ],
)

_tc2 = pl.pallas_call(
    _tc2_body,
    grid=(GRID,),
    in_specs=[_row_spec(), _row_spec(), _col_spec(), _col_spec(),
              _row_spec(), _vec_spec(), _full_spec()],
    out_specs=[_row_spec(), _row_spec()],
    out_shape=[
        jax.ShapeDtypeStruct((NP, D), jnp.float32),
        jax.ShapeDtypeStruct((NP, D), jnp.float32),
    ],
)

_tc3 = pl.pallas_call(
    _tc3_body,
    grid=(GRID,),
    in_specs=[_row_spec(), _row_spec(), _col_spec(), _col_spec(),
              _row_spec(), _vec_spec(), _vec_spec(), _vec_spec()],
    out_specs=_row_spec(),
    out_shape=jax.ShapeDtypeStruct((NP, D), jnp.float32),
)


def kernel(x, edge_index, edge_weight, W1, b1, W2, b2, gamma, beta):
    src = edge_index[0]
    dst = edge_index[1]
    e = src.shape[0]
    pe = EP - e
    src_p = jnp.concatenate(
        [src, jnp.zeros((pe,), jnp.int32)]).reshape(TOTALC, CHUNK)
    dst_p = jnp.concatenate(
        [dst, jnp.zeros((pe,), jnp.int32)]).reshape(TOTALC, CHUNK)
    w_p = jnp.concatenate(
        [edge_weight, jnp.zeros((pe,), jnp.float32)]).reshape(TOTALC, CHUNK)
    xp = jnp.pad(x, ((0, NP - x.shape[0]), (0, 0)))
    e_packed = jnp.stack([src_p, dst_p], axis=1)

    deg_parts = _deg_kernel(e_packed, w_p)
    d0 = deg_parts[0].reshape(NP, 1)
    d1 = deg_parts[1].reshape(NP, 1)

    h1, g1, dis, sn = _tc1(d0, d1, xp, W1)
    p = _msg_kernel(g1, e_packed, w_p)
    h2, g2 = _tc2(p[0], p[1], dis, sn, h1, b1.reshape(1, D), W2)
    q = _msg_kernel(g2, e_packed, w_p)
    out = _tc3(q[0], q[1], dis, sn, h2, b2.reshape(1, D),
               gamma.reshape(1, D), beta.reshape(1, D))
    return out[:N]


# speedup vs baseline: 1.1029x; 1.1029x over previous
import functools

import jax
import jax.numpy as jnp
from jax import lax
from jax.experimental import pallas as pl
from jax.experimental.pallas import tpu as pltpu
from jax.experimental.pallas import tpu_sc as plsc

N = 10000
NP = 10240
D = 128
NC = 2
NS = 16
NW = NC * NS
CHUNK = 64
CHT = 160
QC = 40
Q0 = 6
Q1 = 2
TOTALC = NW * CHT
EP = NW * CHT * CHUNK
TS = NP // NS

_mesh = plsc.VectorSubcoreMesh(core_axis_name="c", subcore_axis_name="s")


@functools.partial(
    pl.kernel,
    out_type=jax.ShapeDtypeStruct((NC, NP), jnp.float32),
    mesh=_mesh,
    scratch_types=[
        pltpu.VMEM_SHARED((NP,), jnp.float32),
        pltpu.VMEM((CHT, 2, CHUNK), jnp.int32),
        pltpu.VMEM((CHT, CHUNK), jnp.float32),
        pltpu.VMEM((TS,), jnp.float32),
    ] + [pltpu.SemaphoreType.DMA for _ in range(8)],
)
def _deg_kernel(e_h, w_h, out_h, acc, ebuf, wbuf, zb,
                d0, d1, d2, d3, d4, d5, d6, d7):
    c = lax.axis_index("c")
    s = lax.axis_index("s")
    wid = c * NS + s
    dsem = [d0, d1, d2, d3, d4, d5, d6, d7]

    def _z(i, carry):
        zb[pl.ds(i * 16, 16)] = jnp.zeros((16,), jnp.float32)
        return carry

    lax.fori_loop(0, TS // 16, _z, 0)
    pltpu.sync_copy(zb, acc.at[pl.ds(s * TS, TS)])
    pltpu.sync_copy(e_h.at[pl.ds(wid * CHT, CHT)], ebuf)
    pltpu.sync_copy(w_h.at[pl.ds(wid * CHT, CHT)], wbuf)
    plsc.subcore_barrier()

    def _fire(l, b):
        pltpu.async_copy(wbuf.at[l], acc.at[ebuf.at[l, 1]], dsem[b], add=True)

    def _drain(l, b):
        pltpu.make_async_copy(wbuf.at[l], acc.at[ebuf.at[l, 1]], dsem[b]).wait()

    for b in range(8):
        _fire(b, b)

    def _grp(g, carry):
        for b in range(8):
            l = 8 + g * 8 + b
            _drain(l - 8, b)
            _fire(l, b)
        return carry

    lax.fori_loop(0, (CHT - 8) // 8, _grp, 0)
    for b in range(8):
        _drain(CHT - 8 + b, b)
    plsc.subcore_barrier()
    sl = pl.ds(s * TS, TS)
    pltpu.sync_copy(acc.at[sl], out_h.at[c, sl])


NBUF = 4


@functools.partial(
    pl.kernel,
    out_type=jax.ShapeDtypeStruct((NC, NP, D), jnp.float32),
    mesh=_mesh,
    scratch_types=[
        pltpu.VMEM_SHARED((NP, D), jnp.float32),
        pltpu.VMEM((QC, 2, CHUNK), jnp.int32),
        pltpu.VMEM((QC, CHUNK), jnp.float32),
    ] + [pltpu.VMEM((CHUNK, D), jnp.float32) for _ in range(NBUF)]
      + [pltpu.SemaphoreType.DMA for _ in range(2 * NBUF)],
)
def _msg_kernel(g_h, e_h, w_h, out_h, acc, idxq, wq, r0, r1, r2, r3,
                g0, g1, g2, g3, s0, s1, s2, s3):
    c = lax.axis_index("c")
    s = lax.axis_index("s")
    nq = jnp.where(c == 0, Q0, Q1)
    qb = jnp.where(c == 0, s * Q0, NS * Q0 + s * Q1)
    rows = [r0, r1, r2, r3]
    gsem = [g0, g1, g2, g3]
    ssem = [s0, s1, s2, s3]

    def _z(i, carry):
        r0[i // 8, pl.ds((i % 8) * 16, 16)] = jnp.zeros((16,), jnp.float32)
        return carry

    lax.fori_loop(0, CHUNK * 8, _z, 0)
    for r in range(TS // CHUNK):
        pltpu.sync_copy(r0, acc.at[pl.ds(s * TS + r * CHUNK, CHUNK)])
    plsc.subcore_barrier()

    def _g_start(l, b):
        pltpu.async_copy(g_h.at[idxq.at[l, 0]], rows[b], gsem[b])

    def _g_wait(l, b):
        pltpu.make_async_copy(g_h.at[idxq.at[l, 0]], rows[b], gsem[b]).wait()

    def _s_start(l, b):
        pltpu.async_copy(rows[b], acc.at[idxq.at[l, 1]], ssem[b], add=True)

    def _s_wait(l, b):
        pltpu.make_async_copy(rows[b], acc.at[idxq.at[l, 1]], ssem[b]).wait()

    def _scale(l, b):
        rb = rows[b]

        def body(g16, c2):
            w16 = wq[l, pl.ds(g16 * 16, 16)]
            for i in range(16):
                co = w16[i]
                e = g16 * 16 + i
                for k in range(8):
                    rb[e, pl.ds(k * 16, 16)] = rb[e, pl.ds(k * 16, 16)] * co
            return c2

        lax.fori_loop(0, CHUNK // 16, body, 0)

    def _quarter(q, carry):
        base = (qb + q) * QC
        pltpu.sync_copy(e_h.at[pl.ds(base, QC)], idxq)
        pltpu.sync_copy(w_h.at[pl.ds(base, QC)], wq)
        _g_start(0, 0)
        _g_start(1, 1)
        for l in (0, 1):
            _g_start(l + 2, l + 2)
            _g_wait(l, l)
            _scale(l, l)
            _s_start(l, l)

        def _group(g, c2):
            for b in range(NBUF):
                l = 2 + g * NBUF + b
                cur = (2 + b) % NBUF
                tgt = b
                _s_wait(l - 2, tgt)
                _g_start(l + 2, tgt)
                _g_wait(l, cur)
                _scale(l, cur)
                _s_start(l, cur)
            return c2

        lax.fori_loop(0, (QC - 4) // NBUF, _group, 0)
        for l in (QC - 2, QC - 1):
            cur = l % NBUF
            _g_wait(l, cur)
            _scale(l, cur)
            _s_start(l, cur)
        for l in (QC - 4, QC - 3, QC - 2, QC - 1):
            _s_wait(l, l % NBUF)
        return carry

    lax.fori_loop(0, nq, _quarter, 0)

    plsc.subcore_barrier()
    for r in range(TS // CHUNK):
        sl = pl.ds(s * TS + r * CHUNK, CHUNK)
        pltpu.sync_copy(acc.at[sl], out_h.at[c, sl])


BLK = 1024
GRID = NP // BLK


def _tc1_body(d0, d1, x, w1, h1, g1, dis, sn):
    deg = d0[...] + d1[...] + 1.0
    di = lax.rsqrt(deg)
    dis[...] = di
    sn[...] = 1.0 / deg
    h = jnp.dot(x[...], w1[...], preferred_element_type=jnp.float32)
    h1[...] = h
    g1[...] = h * di


def _tc2_body(p0, p1, dis, sn, h1, b1, w2, h2, g2):
    z = jnp.maximum(
        (p0[...] + p1[...]) * dis[...] + h1[...] * sn[...] + b1[...], 0.0)
    h = jnp.dot(z, w2[...], preferred_element_type=jnp.float32)
    h2[...] = h
    g2[...] = h * dis[...]


def _tc3_body(q0, q1, dis, sn, h2, b2, gm, bt, out):
    y = (q0[...] + q1[...]) * dis[...] + h2[...] * sn[...] + b2[...]
    mu = jnp.mean(y, axis=-1, keepdims=True)
    yc = y - mu
    var = jnp.mean(yc * yc, axis=-1, keepdims=True)
    out[...] = yc * lax.rsqrt(var + 1e-5) * gm[...] + bt[...]


def _row_spec():
    return pl.BlockSpec((BLK, D), lambda i: (i, 0))


def _col_spec():
    return pl.BlockSpec((BLK, 1), lambda i: (i, 0))


def _full_spec():
    return pl.BlockSpec((D, D), lambda i: (0, 0))


def _vec_spec():
    return pl.BlockSpec((1, D), lambda i: (0, 0))


_tc1 = pl.pallas_call(
    _tc1_body,
    grid=(GRID,),
    in_specs=[_col_spec(), _col_spec(), _row_spec(), _full_spec()],
    out_specs=[_row_spec(), _row_spec(), _col_spec(), _col_spec()],
    out_shape=[
        jax.ShapeDtypeStruct((NP, D), jnp.float32),
        jax.ShapeDtypeStruct((NP, D), jnp.float32),
        jax.ShapeDtypeStruct((NP, 1), jnp.float32),
        jax.ShapeDtypeStruct((NP, 1), jnp.float32),
    ],
)

_tc2 = pl.pallas_call(
    _tc2_body,
    grid=(GRID,),
    in_specs=[_row_spec(), _row_spec(), _col_spec(), _col_spec(),
              _row_spec(), _vec_spec(), _full_spec()],
    out_specs=[_row_spec(), _row_spec()],
    out_shape=[
        jax.ShapeDtypeStruct((NP, D), jnp.float32),
        jax.ShapeDtypeStruct((NP, D), jnp.float32),
    ],
)

_tc3 = pl.pallas_call(
    _tc3_body,
    grid=(GRID,),
    in_specs=[_row_spec(), _row_spec(), _col_spec(), _col_spec(),
              _row_spec(), _vec_spec(), _vec_spec(), _vec_spec()],
    out_specs=_row_spec(),
    out_shape=jax.ShapeDtypeStruct((NP, D), jnp.float32),
)


def kernel(x, edge_index, edge_weight, W1, b1, W2, b2, gamma, beta):
    src = edge_index[0]
    dst = edge_index[1]
    e = src.shape[0]
    pe = EP - e
    src_p = jnp.concatenate(
        [src, jnp.zeros((pe,), jnp.int32)]).reshape(TOTALC, CHUNK)
    dst_p = jnp.concatenate(
        [dst, jnp.zeros((pe,), jnp.int32)]).reshape(TOTALC, CHUNK)
    w_p = jnp.concatenate(
        [edge_weight, jnp.zeros((pe,), jnp.float32)]).reshape(TOTALC, CHUNK)
    xp = jnp.pad(x, ((0, NP - x.shape[0]), (0, 0)))
    e_packed = jnp.stack([src_p, dst_p], axis=1)

    deg_parts = _deg_kernel(e_packed, w_p)
    d0 = deg_parts[0].reshape(NP, 1)
    d1 = deg_parts[1].reshape(NP, 1)

    h1, g1, dis, sn = _tc1(d0, d1, xp, W1)
    p = _msg_kernel(g1, e_packed, w_p)
    h2, g2 = _tc2(p[0], p[1], dis, sn, h1, b1.reshape(1, D), W2)
    q = _msg_kernel(g2, e_packed, w_p)
    out = _tc3(q[0], q[1], dis, sn, h2, b2.reshape(1, D),
               gamma.reshape(1, D), beta.reshape(1, D))
    return out[:N]
